# Initial kernel scaffold; baseline (speedup 1.0000x reference)
import jax
import jax.numpy as jnp
from jax.experimental import pallas as pl


def kernel(x, edge_index, edge_attr, lin_e_W, lin_e_b, W1, b1, W2, b2, gn_weight, gn_bias, gn_mean_scale):
    raise NotImplementedError("write your pallas kernel here")


# speedup vs baseline: 2.7792x; 2.7792x over previous
import functools

import jax
import jax.numpy as jnp
from jax import lax
from jax.experimental import pallas as pl
from jax.experimental.pallas import tpu as pltpu
from jax.experimental.pallas import tpu_sc as plsc

N, E, D, DE = 10000, 320000, 128, 16
NC, NS = 2, 16
NW = NC * NS
EW = E // NW
CHUNK = 128
NFULL = EW // CHUNK
TAIL = EW - NFULL * CHUNK
NPAD = 10240
ROWS_PER_SUB = NPAD // NS


def _proj_body(ea_ref, w_ref, b_ref, out_ref):
    out_ref[...] = (
        jnp.dot(ea_ref[...], w_ref[...], preferred_element_type=jnp.float32)
        + b_ref[...]
    )


def _edge_proj(edge_attr, w, b):
    BE = 8000
    return pl.pallas_call(
        _proj_body,
        grid=(E // BE,),
        in_specs=[
            pl.BlockSpec((BE, DE), lambda i: (i, 0)),
            pl.BlockSpec((DE, D), lambda i: (0, 0)),
            pl.BlockSpec((1, D), lambda i: (0, 0)),
        ],
        out_specs=pl.BlockSpec((BE, D), lambda i: (i, 0)),
        out_shape=jax.ShapeDtypeStruct((E, D), jnp.float32),
    )(edge_attr, w, b.reshape(1, D))


def _sc_aggregate(x, src, dst, eproj, zeros):
    mesh = plsc.VectorSubcoreMesh(core_axis_name="c", subcore_axis_name="s")

    @functools.partial(
        pl.kernel,
        mesh=mesh,
        out_type=jax.ShapeDtypeStruct((NC, NPAD, D), jnp.float32),
        scratch_types=[
            pltpu.VMEM((CHUNK,), jnp.int32),
            pltpu.VMEM((CHUNK,), jnp.int32),
            pltpu.VMEM((CHUNK, D), jnp.float32),
            pltpu.VMEM((CHUNK, D), jnp.float32),
            pltpu.VMEM((TAIL,), jnp.int32),
            pltpu.VMEM((TAIL,), jnp.int32),
            pltpu.VMEM((TAIL, D), jnp.float32),
            pltpu.VMEM((TAIL, D), jnp.float32),
            pltpu.VMEM_SHARED((NPAD, D), jnp.float32),
            pltpu.SemaphoreType.DMA,
        ],
    )
    def k(x_hbm, src_hbm, dst_hbm, ep_hbm, z_hbm, out_hbm,
          srcv, dstv, xg, ev, srcvt, dstvt, xgt, evt, aggsh, sem):
        cid = lax.axis_index("c")
        sid = lax.axis_index("s")
        wid = sid * NC + cid
        pltpu.sync_copy(
            z_hbm.at[pl.ds(sid * ROWS_PER_SUB, ROWS_PER_SUB)],
            aggsh.at[pl.ds(sid * ROWS_PER_SUB, ROWS_PER_SUB)],
        )
        plsc.subcore_barrier()
        ebase = wid * EW

        def do_chunk(base, sv, dv, xb, eb, nsz):
            pltpu.sync_copy(src_hbm.at[pl.ds(base, nsz)], sv)
            pltpu.sync_copy(dst_hbm.at[pl.ds(base, nsz)], dv)
            pltpu.async_copy(x_hbm.at[sv], xb, sem).wait()
            pltpu.sync_copy(ep_hbm.at[pl.ds(base, nsz)], eb)

            @pl.loop(0, nsz)
            def _(i):
                for j in range(D // 16):
                    sl = pl.ds(j * 16, 16)
                    eb[i, sl] = jnp.maximum(eb[i, sl] + xb[i, sl], 0.0)

            pltpu.sync_copy(eb, aggsh.at[dv], add=True)

        @pl.loop(0, NFULL)
        def _(c):
            do_chunk(ebase + c * CHUNK, srcv, dstv, xg, ev, CHUNK)

        if TAIL:
            do_chunk(ebase + NFULL * CHUNK, srcvt, dstvt, xgt, evt, TAIL)

        plsc.subcore_barrier()
        pltpu.sync_copy(
            aggsh.at[pl.ds(sid * ROWS_PER_SUB, ROWS_PER_SUB)],
            out_hbm.at[cid, pl.ds(sid * ROWS_PER_SUB, ROWS_PER_SUB)],
        )

    return k(x, src, dst, eproj, zeros)


def _mlp_norm_body(x_ref, p_ref, w1_ref, b1_ref, w2_ref, b2_ref,
                   gw_ref, gb_ref, gs_ref, out_ref):
    h = x_ref[...] + p_ref[0] + p_ref[1]
    a = jnp.maximum(
        jnp.dot(h, w1_ref[...], preferred_element_type=jnp.float32)
        + b1_ref[...], 0.0)
    t = (jnp.dot(a, w2_ref[...], preferred_element_type=jnp.float32)
         + b2_ref[...])
    m = jnp.mean(t, axis=0, keepdims=True)
    c = t - gs_ref[...] * m
    v = jnp.mean(c * c, axis=0, keepdims=True)
    out_ref[...] = jnp.maximum(
        gw_ref[...] * c * lax.rsqrt(v + 1e-5) + gb_ref[...], 0.0)


def _mlp_norm(x, partials, W1, b1, W2, b2, gn_weight, gn_bias, gn_mean_scale):
    return pl.pallas_call(
        _mlp_norm_body,
        out_shape=jax.ShapeDtypeStruct((N, D), jnp.float32),
    )(x, partials, W1, b1.reshape(1, D), W2, b2.reshape(1, D),
      gn_weight.reshape(1, D), gn_bias.reshape(1, D),
      gn_mean_scale.reshape(1, D))


def kernel(x, edge_index, edge_attr, lin_e_W, lin_e_b, W1, b1, W2, b2,
           gn_weight, gn_bias, gn_mean_scale):
    eproj = _edge_proj(edge_attr, lin_e_W, lin_e_b)
    src = edge_index[0]
    dst = edge_index[1]
    zeros = jnp.zeros((NPAD, D), jnp.float32)
    partials = _sc_aggregate(x, src, dst, eproj, zeros)
    return _mlp_norm(x, partials[:, :N, :], W1, b1, W2, b2,
                     gn_weight, gn_bias, gn_mean_scale)


# speedup vs baseline: 3.0627x; 1.1020x over previous
import functools

import jax
import jax.numpy as jnp
from jax import lax
from jax.experimental import pallas as pl
from jax.experimental.pallas import tpu as pltpu
from jax.experimental.pallas import tpu_sc as plsc

N, E, D, DE = 10000, 320000, 128, 16
NC, NS = 2, 16
NW = NC * NS
CHUNK = 80
NCHUNK = 126
EPAD = NW * NCHUNK * CHUNK
NPAD = 10240
ROWS_PER_SUB = NPAD // NS


def _proj_body(ea_ref, w_ref, b_ref, out_ref):
    out_ref[...] = (
        jnp.dot(ea_ref[...], w_ref[...], preferred_element_type=jnp.float32)
        + b_ref[...]
    )


def _edge_proj(edge_attr, w, b):
    BE = 7680
    return pl.pallas_call(
        _proj_body,
        grid=(EPAD // BE,),
        in_specs=[
            pl.BlockSpec((BE, DE), lambda i: (i, 0)),
            pl.BlockSpec((DE, D), lambda i: (0, 0)),
            pl.BlockSpec((1, D), lambda i: (0, 0)),
        ],
        out_specs=pl.BlockSpec((BE, D), lambda i: (i, 0)),
        out_shape=jax.ShapeDtypeStruct((EPAD, D), jnp.float32),
    )(edge_attr, w, b.reshape(1, D))


def _sc_aggregate(x, src, dst, eproj, zeros):
    mesh = plsc.VectorSubcoreMesh(core_axis_name="c", subcore_axis_name="s")

    @functools.partial(
        pl.kernel,
        mesh=mesh,
        out_type=jax.ShapeDtypeStruct((NC, NPAD, D), jnp.float32),
        scratch_types=[
            pltpu.VMEM((CHUNK,), jnp.int32),
            pltpu.VMEM((CHUNK,), jnp.int32),
            pltpu.VMEM((CHUNK,), jnp.int32),
            pltpu.VMEM((CHUNK,), jnp.int32),
            pltpu.VMEM((CHUNK, D), jnp.float32),
            pltpu.VMEM((CHUNK, D), jnp.float32),
            pltpu.VMEM((CHUNK, D), jnp.float32),
            pltpu.VMEM((CHUNK, D), jnp.float32),
            pltpu.VMEM_SHARED((NPAD, D), jnp.float32),
            pltpu.SemaphoreType.DMA,
            pltpu.SemaphoreType.DMA,
            pltpu.SemaphoreType.DMA,
            pltpu.SemaphoreType.DMA,
        ],
    )
    def k(x_hbm, src_hbm, dst_hbm, ep_hbm, z_hbm, out_hbm,
          sv0, dv0, sv1, dv1, xg0, ev0, xg1, ev1, aggsh,
          isem0, isem1, sem0, sem1):
        cid = lax.axis_index("c")
        sid = lax.axis_index("s")
        wid = sid * NC + cid
        pltpu.sync_copy(
            z_hbm.at[pl.ds(sid * ROWS_PER_SUB, ROWS_PER_SUB)],
            aggsh.at[pl.ds(sid * ROWS_PER_SUB, ROWS_PER_SUB)],
        )
        plsc.subcore_barrier()
        ebase = wid * NCHUNK * CHUNK
        bufs = ((sv0, dv0, xg0, ev0, isem0, sem0),
                (sv1, dv1, xg1, ev1, isem1, sem1))

        def idx_copies(c, b):
            sv, dv, _, _, isem, _ = bufs[b]
            off = ebase + c * CHUNK
            return (
                pltpu.make_async_copy(src_hbm.at[pl.ds(off, CHUNK)], sv, isem),
                pltpu.make_async_copy(dst_hbm.at[pl.ds(off, CHUNK)], dv, isem),
            )

        def data_copies(c, b):
            sv, _, xb, eb, _, sem = bufs[b]
            return (
                pltpu.make_async_copy(x_hbm.at[sv], xb, sem),
                pltpu.make_async_copy(
                    ep_hbm.at[pl.ds(ebase + c * CHUNK, CHUNK)], eb, sem),
            )

        def start(copies):
            for cp in copies:
                cp.start()

        def wait(copies):
            for cp in copies:
                cp.wait()

        start(idx_copies(0, 0))
        start(idx_copies(1, 1))
        wait(idx_copies(0, 0))
        start(data_copies(0, 0))

        @pl.loop(0, NCHUNK, step=2)
        def _(g):
            for b in (0, 1):
                c = g + b
                _, dv, xb, eb, _, _ = bufs[b]

                wait(data_copies(c, b))

                @pl.when(c + 1 < NCHUNK)
                def _():
                    wait(idx_copies(c + 1, 1 - b))
                    start(data_copies(c + 1, 1 - b))

                @pl.loop(0, CHUNK)
                def _(i):
                    for j in range(D // 16):
                        sl = pl.ds(j * 16, 16)
                        eb[i, sl] = jnp.maximum(eb[i, sl] + xb[i, sl], 0.0)

                pltpu.sync_copy(eb, aggsh.at[dv], add=True)

                @pl.when(c + 2 < NCHUNK)
                def _():
                    start(idx_copies(c + 2, b))

        plsc.subcore_barrier()
        pltpu.sync_copy(
            aggsh.at[pl.ds(sid * ROWS_PER_SUB, ROWS_PER_SUB)],
            out_hbm.at[cid, pl.ds(sid * ROWS_PER_SUB, ROWS_PER_SUB)],
        )

    return k(x, src, dst, eproj, zeros)


def _mlp_norm_body(x_ref, p_ref, w1_ref, b1_ref, w2_ref, b2_ref,
                   gw_ref, gb_ref, gs_ref, out_ref):
    h = x_ref[...] + p_ref[0] + p_ref[1]
    a = jnp.maximum(
        jnp.dot(h, w1_ref[...], preferred_element_type=jnp.float32)
        + b1_ref[...], 0.0)
    t = (jnp.dot(a, w2_ref[...], preferred_element_type=jnp.float32)
         + b2_ref[...])
    m = jnp.mean(t, axis=0, keepdims=True)
    c = t - gs_ref[...] * m
    v = jnp.mean(c * c, axis=0, keepdims=True)
    out_ref[...] = jnp.maximum(
        gw_ref[...] * c * lax.rsqrt(v + 1e-5) + gb_ref[...], 0.0)


def _mlp_norm(x, partials, W1, b1, W2, b2, gn_weight, gn_bias, gn_mean_scale):
    return pl.pallas_call(
        _mlp_norm_body,
        out_shape=jax.ShapeDtypeStruct((N, D), jnp.float32),
    )(x, partials, W1, b1.reshape(1, D), W2, b2.reshape(1, D),
      gn_weight.reshape(1, D), gn_bias.reshape(1, D),
      gn_mean_scale.reshape(1, D))


def kernel(x, edge_index, edge_attr, lin_e_W, lin_e_b, W1, b1, W2, b2,
           gn_weight, gn_bias, gn_mean_scale):
    pad = EPAD - E
    ea_p = jnp.pad(edge_attr, ((0, pad), (0, 0)))
    src_p = jnp.pad(edge_index[0], (0, pad))
    dst_p = jnp.pad(edge_index[1], (0, pad), constant_values=N)
    eproj = _edge_proj(ea_p, lin_e_W, lin_e_b)
    zeros = jnp.zeros((NPAD, D), jnp.float32)
    partials = _sc_aggregate(x, src_p, dst_p, eproj, zeros)
    return _mlp_norm(x, partials[:, :N, :], W1, b1, W2, b2,
                     gn_weight, gn_bias, gn_mean_scale)


# speedup vs baseline: 4.0686x; 1.3285x over previous
import functools

import jax
import jax.numpy as jnp
from jax import lax
from jax.experimental import pallas as pl
from jax.experimental.pallas import tpu as pltpu
from jax.experimental.pallas import tpu_sc as plsc

N, E, D, DE = 10000, 320000, 128, 16
NC, NS = 2, 16
NW = NC * NS
CHUNK = 64
NCHUNK = 156
NTAIL = E // CHUNK - NW * NCHUNK
NPAD = 10240
ROWS_PER_SUB = NPAD // NS


def _proj_body(ea_ref, w_ref, b_ref, out_ref):
    out_ref[...] = (
        jnp.dot(ea_ref[...], w_ref[...], preferred_element_type=jnp.float32)
        + b_ref[...]
    )


def _edge_proj(edge_attr, w, b):
    BE = 8000
    return pl.pallas_call(
        _proj_body,
        grid=(E // BE,),
        in_specs=[
            pl.BlockSpec((BE, DE), lambda i: (i, 0)),
            pl.BlockSpec((DE, D), lambda i: (0, 0)),
            pl.BlockSpec((1, D), lambda i: (0, 0)),
        ],
        out_specs=pl.BlockSpec((BE, D), lambda i: (i, 0)),
        out_shape=jax.ShapeDtypeStruct((E, D), jnp.float32),
    )(edge_attr, w, b.reshape(1, D))


def _sc_aggregate(x, src, dst, eproj, zeros):
    mesh = plsc.VectorSubcoreMesh(core_axis_name="c", subcore_axis_name="s")

    @functools.partial(
        pl.kernel,
        mesh=mesh,
        out_type=jax.ShapeDtypeStruct((NC, NPAD, D), jnp.float32),
        scratch_types=(
            [pltpu.VMEM((CHUNK,), jnp.int32)] * 2
            + [pltpu.VMEM((CHUNK,), jnp.int32)] * 3
            + [pltpu.VMEM((CHUNK, D), jnp.float32)] * 2
            + [pltpu.VMEM((CHUNK, D), jnp.float32)] * 3
            + [pltpu.VMEM_SHARED((NPAD, D), jnp.float32)]
            + [pltpu.SemaphoreType.DMA] * 13
        ),
    )
    def k(x_hbm, src_hbm, dst_hbm, ep_hbm, z_hbm, out_hbm,
          sv0, sv1, dv0, dv1, dv2, xg0, xg1, ev0, ev1, ev2, aggsh,
          gsem0, gsem1, esem0, esem1, esem2, ssem0, ssem1, ssem2,
          isem0, isem1, dsem0, dsem1, dsem2):
        sv = (sv0, sv1)
        dv = (dv0, dv1, dv2)
        xg = (xg0, xg1)
        ev = (ev0, ev1, ev2)
        gsem = (gsem0, gsem1)
        esem = (esem0, esem1, esem2)
        ssem = (ssem0, ssem1, ssem2)
        isem = (isem0, isem1)
        dsem = (dsem0, dsem1, dsem2)
        cid = lax.axis_index("c")
        sid = lax.axis_index("s")
        wid = sid * NC + cid
        pltpu.sync_copy(
            z_hbm.at[pl.ds(sid * ROWS_PER_SUB, ROWS_PER_SUB)],
            aggsh.at[pl.ds(sid * ROWS_PER_SUB, ROWS_PER_SUB)],
        )
        plsc.subcore_barrier()
        ebase = wid * NCHUNK * CHUNK

        def src_cp(c, s2):
            return pltpu.make_async_copy(
                src_hbm.at[pl.ds(ebase + c * CHUNK, CHUNK)],
                sv[s2], isem[s2])

        def dst_cp(c, s3):
            return pltpu.make_async_copy(
                dst_hbm.at[pl.ds(ebase + c * CHUNK, CHUNK)],
                dv[s3], dsem[s3])

        def gather_cp(s2):
            return pltpu.make_async_copy(
                x_hbm.at[sv[s2]], xg[s2], gsem[s2])

        def ep_cp(c, s3):
            return pltpu.make_async_copy(
                ep_hbm.at[pl.ds(ebase + c * CHUNK, CHUNK)],
                ev[s3], esem[s3])

        def scat_cp(s3):
            return pltpu.make_async_copy(
                ev[s3], aggsh.at[dv[s3]], ssem[s3])

        def compute(s2, s3):
            xb, eb = xg[s2], ev[s3]

            @pl.loop(0, CHUNK)
            def _(i):
                for j in range(D // 16):
                    sl = pl.ds(j * 16, 16)
                    eb[i, sl] = jnp.maximum(eb[i, sl] + xb[i, sl], 0.0)

        src_cp(0, 0).start()
        dst_cp(0, 0).start()
        src_cp(1, 1).start()
        src_cp(0, 0).wait()
        gather_cp(0).start()
        ep_cp(0, 0).start()
        dst_cp(1, 1).start()

        @pl.loop(0, NCHUNK, step=6)
        def _(g):
            for u in range(6):
                c = g + u
                s2, s3 = u % 2, u % 3
                n2, n3 = (u + 1) % 2, (u + 1) % 3
                p3 = (u - 1) % 3

                gather_cp(s2).wait()
                ep_cp(c, s3).wait()

                @pl.when(c + 1 < NCHUNK)
                def _():
                    src_cp(c + 1, n2).wait()
                    gather_cp(n2).start()
                    ep_cp(c + 1, n3).start()

                    @pl.when(c >= 1)
                    def _():
                        dst_cp(c + 1, n3).start()

                @pl.when(c + 2 < NCHUNK)
                def _():
                    src_cp(c + 2, s2).start()

                compute(s2, s3)

                dst_cp(c, s3).wait()

                @pl.when(c >= 1)
                def _():
                    scat_cp(p3).wait()

                scat_cp(s3).start(add=True)

        scat_cp((NCHUNK - 1) % 3).wait()

        @pl.when(wid < NTAIL)
        def _():
            toff = (NW * NCHUNK + wid) * CHUNK

            def t_src():
                return pltpu.make_async_copy(
                    src_hbm.at[pl.ds(toff, CHUNK)], sv[0], isem[0])

            def t_dst():
                return pltpu.make_async_copy(
                    dst_hbm.at[pl.ds(toff, CHUNK)], dv[0], dsem[0])

            def t_ep():
                return pltpu.make_async_copy(
                    ep_hbm.at[pl.ds(toff, CHUNK)], ev[0], esem[0])

            t_src().start()
            t_dst().start()
            t_src().wait()
            gather_cp(0).start()
            t_ep().start()
            gather_cp(0).wait()
            t_ep().wait()
            compute(0, 0)
            t_dst().wait()
            scat_cp(0).start(add=True)
            scat_cp(0).wait()

        plsc.subcore_barrier()
        pltpu.sync_copy(
            aggsh.at[pl.ds(sid * ROWS_PER_SUB, ROWS_PER_SUB)],
            out_hbm.at[cid, pl.ds(sid * ROWS_PER_SUB, ROWS_PER_SUB)],
        )

    return k(x, src, dst, eproj, zeros)


def _mlp_norm_body(x_ref, p_ref, w1_ref, b1_ref, w2_ref, b2_ref,
                   gw_ref, gb_ref, gs_ref, out_ref):
    h = x_ref[...] + p_ref[0] + p_ref[1]
    a = jnp.maximum(
        jnp.dot(h, w1_ref[...], preferred_element_type=jnp.float32)
        + b1_ref[...], 0.0)
    t = (jnp.dot(a, w2_ref[...], preferred_element_type=jnp.float32)
         + b2_ref[...])
    m = jnp.mean(t, axis=0, keepdims=True)
    c = t - gs_ref[...] * m
    v = jnp.mean(c * c, axis=0, keepdims=True)
    out_ref[...] = jnp.maximum(
        gw_ref[...] * c * lax.rsqrt(v + 1e-5) + gb_ref[...], 0.0)


def _mlp_norm(x, partials, W1, b1, W2, b2, gn_weight, gn_bias, gn_mean_scale):
    return pl.pallas_call(
        _mlp_norm_body,
        out_shape=jax.ShapeDtypeStruct((N, D), jnp.float32),
    )(x, partials, W1, b1.reshape(1, D), W2, b2.reshape(1, D),
      gn_weight.reshape(1, D), gn_bias.reshape(1, D),
      gn_mean_scale.reshape(1, D))


def kernel(x, edge_index, edge_attr, lin_e_W, lin_e_b, W1, b1, W2, b2,
           gn_weight, gn_bias, gn_mean_scale):
    eproj = _edge_proj(edge_attr, lin_e_W, lin_e_b)
    zeros = jnp.zeros((NPAD, D), jnp.float32)
    partials = _sc_aggregate(x, edge_index[0], edge_index[1], eproj, zeros)
    return _mlp_norm(x, partials[:, :N, :], W1, b1, W2, b2,
                     gn_weight, gn_bias, gn_mean_scale)
